# baseline (speedup 1.0000x reference)
import functools

import jax
import jax.numpy as jnp
from jax import lax
from jax.experimental import pallas as pl
from jax.experimental.pallas import tpu as pltpu
from jax.experimental.pallas import tpu_sc as plsc

_COMMIT = 0.1
_QUANT = 0.2


def _argmin_body(n_chunks, kc, f_ref, e_ref, fn2_ref, en2_ref, idx_ref, loss_ref):
    f = f_ref[...]
    r, d = f.shape
    f2 = f * 2.0
    fn2 = fn2_ref[...].reshape(r, 1)
    dn = (((1,), (1,)), ((), ()))

    lane_f = lax.broadcasted_iota(jnp.int32, (r, kc), 1).astype(jnp.float32)
    bv = jnp.full((r, 1), jnp.inf, dtype=jnp.float32)
    bi = jnp.zeros((r, 1), dtype=jnp.int32)
    lv = bv
    for j in range(n_chunks):
        e = e_ref[pl.ds(j * kc, kc), :]
        en2 = en2_ref[:, pl.ds(j * kc, kc)]
        mm = lax.dot_general(f2, e, dn)
        dist = (fn2 + en2) - mm
        cmin = jnp.min(dist, axis=1, keepdims=True)
        cidx_f = jnp.min(
            jnp.where(dist == cmin, lane_f, jnp.inf), axis=1, keepdims=True)
        cidx = cidx_f.astype(jnp.int32) + j * kc
        upd = (cmin < bv) | ((cmin == bv) & (cidx < bi))
        bv = jnp.where(upd, cmin, bv).astype(jnp.bfloat16).astype(jnp.float32)
        bi = jnp.where(upd, cidx, bi)
        lv = jnp.where(upd, cmin, lv)
    idx_ref[...] = bi.reshape(r)
    mean = lv * (1.0 / d)
    loss_ref[...] = (mean * _COMMIT + mean * _QUANT).reshape(r)


def _argmin_call(flat, emb, fn2, en2, row_block=512, k_chunk=4096):
    n, d = flat.shape
    k = emb.shape[0]
    grid = (n // row_block,)
    body = functools.partial(_argmin_body, k // k_chunk, k_chunk)
    return pl.pallas_call(
        body,
        grid=grid,
        in_specs=[
            pl.BlockSpec((row_block, d), lambda i: (i, 0)),
            pl.BlockSpec((k, d), lambda i: (0, 0)),
            pl.BlockSpec((1, row_block), lambda i: (0, i)),
            pl.BlockSpec((1, k), lambda i: (0, 0)),
        ],
        out_specs=[
            pl.BlockSpec((row_block,), lambda i: (i,)),
            pl.BlockSpec((row_block,), lambda i: (i,)),
        ],
        out_shape=[
            jax.ShapeDtypeStruct((n,), jnp.int32),
            jax.ShapeDtypeStruct((n,), jnp.float32),
        ],
    )(flat, emb, fn2, en2)


def _sc_gather(emb, idx2d, out_shape):
    b, hw, d = out_shape
    n = b * hw
    info = plsc.get_sparse_core_info()
    nw = info.num_cores * info.num_subcores
    rows_per_w = n // nw
    ic = 128
    nc_loc = rows_per_w // ic
    w_per_b = hw // rows_per_w
    mesh = plsc.VectorSubcoreMesh(core_axis_name="c", subcore_axis_name="s")

    @functools.partial(
        pl.kernel, mesh=mesh,
        out_type=jax.ShapeDtypeStruct((b, hw, d), jnp.float32),
        compiler_params=pltpu.CompilerParams(use_tc_tiling_on_sc=False),
        scratch_types=[
            pltpu.VMEM((nc_loc, ic), jnp.int32),
            pltpu.VMEM((rows_per_w, d), jnp.float32),
            pltpu.SemaphoreType.DMA,
        ],
    )
    def gk(table_hbm, idx_hbm, out_hbm, idx_v, rows_v, sem):
        wid = lax.axis_index("s") * info.num_cores + lax.axis_index("c")
        pltpu.sync_copy(idx_hbm.at[pl.ds(wid * nc_loc, nc_loc)], idx_v)
        copies = [
            pltpu.async_copy(
                table_hbm.at[idx_v.at[c]],
                rows_v.at[pl.ds(c * ic, ic)], sem)
            for c in range(nc_loc)
        ]
        for cp in copies:
            cp.wait()
        pltpu.sync_copy(
            rows_v,
            out_hbm.at[wid // w_per_b,
                       pl.ds((wid % w_per_b) * rows_per_w, rows_per_w)])

    return gk(emb, idx2d)


def kernel(h, embeddings):
    flat = h.reshape(-1, h.shape[-1])
    fn2 = (flat ** 2).sum(axis=-1)[None, :]
    en2 = (embeddings ** 2).sum(axis=-1)[None, :]
    idx, loss = _argmin_call(flat, embeddings, fn2, en2)
    idx_rows = idx.reshape(-1, 128)
    quantized = _sc_gather(embeddings, idx_rows, h.shape)
    return (quantized, idx[:, None], loss)


# speedup vs baseline: 1.0328x; 1.0328x over previous
import functools

import jax
import jax.numpy as jnp
from jax import lax
from jax.experimental import pallas as pl
from jax.experimental.pallas import tpu as pltpu
from jax.experimental.pallas import tpu_sc as plsc

_COMMIT = 0.1
_QUANT = 0.2


def _argmin_body(n_chunks, kc, f_ref, e_ref, fn2_ref, en2_ref, idx_ref, loss_ref):
    f = f_ref[...]
    r, d = f.shape
    f2 = f * 2.0
    fn2 = fn2_ref[...].reshape(r, 1)
    dn = (((1,), (1,)), ((), ()))

    lane_f = lax.broadcasted_iota(jnp.int32, (r, kc), 1).astype(jnp.float32)
    bv = jnp.full((r, 1), jnp.inf, dtype=jnp.float32)
    bi = jnp.zeros((r, 1), dtype=jnp.int32)
    lv = bv
    for j in range(n_chunks):
        e = e_ref[pl.ds(j * kc, kc), :]
        en2 = en2_ref[:, pl.ds(j * kc, kc)]
        mm = lax.dot_general(f2, e, dn)
        dist = (fn2 + en2) - mm
        cmin = jnp.min(dist, axis=1, keepdims=True)
        cidx_f = jnp.min(
            jnp.where(dist == cmin, lane_f, jnp.inf), axis=1, keepdims=True)
        cidx = cidx_f.astype(jnp.int32) + j * kc
        upd = (cmin < bv) | ((cmin == bv) & (cidx < bi))
        bv = jnp.where(upd, cmin, bv).astype(jnp.bfloat16).astype(jnp.float32)
        bi = jnp.where(upd, cidx, bi)
        lv = jnp.where(upd, cmin, lv)
    idx_ref[...] = bi.reshape(r)
    mean = lv * (1.0 / d)
    loss_ref[...] = (mean * _COMMIT + mean * _QUANT).reshape(r)


def _argmin_call(flat, emb, fn2, en2, row_block=2048, k_chunk=4096):
    n, d = flat.shape
    k = emb.shape[0]
    grid = (n // row_block,)
    body = functools.partial(_argmin_body, k // k_chunk, k_chunk)
    return pl.pallas_call(
        body,
        grid=grid,
        in_specs=[
            pl.BlockSpec((row_block, d), lambda i: (i, 0)),
            pl.BlockSpec((k, d), lambda i: (0, 0)),
            pl.BlockSpec((1, row_block), lambda i: (0, i)),
            pl.BlockSpec((1, k), lambda i: (0, 0)),
        ],
        out_specs=[
            pl.BlockSpec((row_block,), lambda i: (i,)),
            pl.BlockSpec((row_block,), lambda i: (i,)),
        ],
        out_shape=[
            jax.ShapeDtypeStruct((n,), jnp.int32),
            jax.ShapeDtypeStruct((n,), jnp.float32),
        ],
    )(flat, emb, fn2, en2)


def _sc_gather(emb, idx2d, out_shape):
    b, hw, d = out_shape
    n = b * hw
    info = plsc.get_sparse_core_info()
    nw = info.num_cores * info.num_subcores
    rows_per_w = n // nw
    ic = 128
    nc_loc = rows_per_w // ic
    w_per_b = hw // rows_per_w
    mesh = plsc.VectorSubcoreMesh(core_axis_name="c", subcore_axis_name="s")

    @functools.partial(
        pl.kernel, mesh=mesh,
        out_type=jax.ShapeDtypeStruct((b, hw, d), jnp.float32),
        compiler_params=pltpu.CompilerParams(use_tc_tiling_on_sc=False),
        scratch_types=[
            pltpu.VMEM((nc_loc, ic), jnp.int32),
            pltpu.VMEM((rows_per_w, d), jnp.float32),
            pltpu.SemaphoreType.DMA,
        ],
    )
    def gk(table_hbm, idx_hbm, out_hbm, idx_v, rows_v, sem):
        wid = lax.axis_index("s") * info.num_cores + lax.axis_index("c")
        pltpu.sync_copy(idx_hbm.at[pl.ds(wid * nc_loc, nc_loc)], idx_v)
        copies = [
            pltpu.async_copy(
                table_hbm.at[idx_v.at[c]],
                rows_v.at[pl.ds(c * ic, ic)], sem)
            for c in range(nc_loc)
        ]
        for cp in copies:
            cp.wait()
        pltpu.sync_copy(
            rows_v,
            out_hbm.at[wid // w_per_b,
                       pl.ds((wid % w_per_b) * rows_per_w, rows_per_w)])

    return gk(emb, idx2d)


def kernel(h, embeddings):
    flat = h.reshape(-1, h.shape[-1])
    fn2 = (flat ** 2).sum(axis=-1)[None, :]
    en2 = (embeddings ** 2).sum(axis=-1)[None, :]
    idx, loss = _argmin_call(flat, embeddings, fn2, en2)
    idx_rows = idx.reshape(-1, 128)
    quantized = _sc_gather(embeddings, idx_rows, h.shape)
    return (quantized, idx[:, None], loss)
